# Initial kernel scaffold; baseline (speedup 1.0000x reference)
import jax
import jax.numpy as jnp
from jax.experimental import pallas as pl


def kernel(x, edge_index, edge_attr, line_graph_edge_index, line_graph_edge_attr, u, batch, params):
    raise NotImplementedError("write your pallas kernel here")


# speedup vs baseline: 1.0125x; 1.0125x over previous
import jax
import jax.numpy as jnp
from jax.experimental import pallas as pl


def _linear(p, x):
    return x @ p["w"] + p["b"]


def _silu(x):
    return x * jax.nn.sigmoid(x)


def _egc_node(p, x, src, dst, edge_attr):
    combined = jnp.concatenate([x[dst], x[src], edge_attr], axis=-1)
    gate = jax.nn.sigmoid(_linear(p["gate2"], _silu(_linear(p["gate1"], combined))))
    msg = _linear(p["node3"], _silu(_linear(p["node2"], _silu(_linear(p["node1"], combined)))))
    x_new = x + jnp.zeros_like(x).at[dst].add(gate * msg)
    return x_new


def _egc_edge_update(p, x_new, src, dst, edge_attr):
    ec = jnp.concatenate([x_new[src], x_new[dst], edge_attr], axis=-1)
    return edge_attr + _linear(p["edge2"], _silu(_linear(p["edge1"], ec)))


def _final_mlp_kernel(comb_ref, w1_ref, b1_ref, w2_ref, b2_ref, out_ref):
    z = comb_ref[...] @ w1_ref[...] + b1_ref[...]
    z = z * (1.0 / (1.0 + jnp.exp(-z)))
    out_ref[...] = z @ w2_ref[...] + b2_ref[...]


def kernel(x, edge_index, edge_attr, line_graph_edge_index, line_graph_edge_attr, u, batch, params):
    src = edge_index[0]
    dst = edge_index[1]
    lsrc = line_graph_edge_index[0]
    ldst = line_graph_edge_index[1]

    h = _linear(params["node_embed"], x)
    e = _linear(params["edge_embed"], edge_attr)
    uu = u[None, :] if u.ndim == 1 else u
    u_embed = _linear(params["global_embed"], uu)

    l0, l1 = params["layers"][0], params["layers"][1]
    h = _egc_node(l0["atom"], h, src, dst, e)
    e = _egc_edge_update(l0["atom"], h, src, dst, e)
    e = _egc_node(l0["edge"], e, lsrc, ldst, line_graph_edge_attr)
    h = _egc_node(l1["atom"], h, src, dst, e)

    pool = jnp.mean(h, axis=0, keepdims=True)
    comb = jnp.concatenate([pool, u_embed], axis=-1)

    out = pl.pallas_call(
        _final_mlp_kernel,
        out_shape=jax.ShapeDtypeStruct((1, 1), jnp.float32),
    )(comb, params["out1"]["w"], params["out1"]["b"], params["out2"]["w"], params["out2"]["b"])
    return out


# speedup vs baseline: 1.2123x; 1.1972x over previous
import functools

import jax
import jax.numpy as jnp
from jax import lax
from jax.experimental import pallas as pl
from jax.experimental.pallas import tpu as pltpu
from jax.experimental.pallas import tpu_sc as plsc

N, E = 10000, 160000
D = 64
EP = 163840
NW = 32
PER_W = EP // NW
CH = 512
NCHUNK = PER_W // CH

NPAD = 10240
FS = 16
RHALF = 81920
SP_ROWS = 82944
DUMP = 82900

_MESH = dict(core_axis_name="c", subcore_axis_name="s")
_SC_PARAMS = pltpu.CompilerParams(use_tc_tiling_on_sc=False)


def _zero_fill(zbuf):
    z = jnp.zeros((16,), jnp.float32)
    nv = zbuf.shape[1] // 16
    def row(r, _):
        for v in range(nv):
            zbuf[r, pl.ds(v * 16, 16)] = z
        return 0
    lax.fori_loop(0, zbuf.shape[0], row, 0)


def _make_gather2():
    mesh = plsc.VectorSubcoreMesh(**_MESH)

    @functools.partial(
        pl.kernel,
        out_type=jax.ShapeDtypeStruct((2, EP, D), jnp.float32),
        mesh=mesh,
        scratch_types=[
            pltpu.VMEM((2, CH), jnp.int32),
            pltpu.VMEM((2, CH, D), jnp.float32),
            pltpu.SemaphoreType.DMA,
            pltpu.SemaphoreType.DMA,
        ],
        compiler_params=_SC_PARAMS,
    )
    def k(table, idx2, out, idxv, gbuf, sem0, sem1):
        wid = lax.axis_index("s") * 2 + lax.axis_index("c")
        base = wid * PER_W

        def chunk(c, _):
            off = base + c * CH
            pltpu.sync_copy(idx2.at[0, pl.ds(off, CH)], idxv.at[0])
            pltpu.sync_copy(idx2.at[1, pl.ds(off, CH)], idxv.at[1])
            cp0 = pltpu.async_copy(table.at[idxv.at[0]], gbuf.at[0], sem0)
            cp1 = pltpu.async_copy(table.at[idxv.at[1]], gbuf.at[1], sem1)
            cp0.wait()
            cp1.wait()
            pltpu.sync_copy(gbuf.at[0], out.at[0, pl.ds(off, CH)])
            pltpu.sync_copy(gbuf.at[1], out.at[1, pl.ds(off, CH)])
            return 0

        lax.fori_loop(0, NCHUNK, chunk, 0)

    return k


def _scatter_small(msgs, idx):
    mesh = plsc.VectorSubcoreMesh(**_MESH)

    @functools.partial(
        pl.kernel,
        out_type=jax.ShapeDtypeStruct((2, NPAD, D), jnp.float32),
        mesh=mesh,
        scratch_types=[
            pltpu.VMEM((CH,), jnp.int32),
            pltpu.VMEM((CH, D), jnp.float32),
            pltpu.VMEM((64, D), jnp.float32),
            pltpu.VMEM_SHARED((NPAD, D), jnp.float32),
        ],
        compiler_params=_SC_PARAMS,
    )
    def k(msg, ind, out, idxv, mbuf, zbuf, acc):
        cid = lax.axis_index("c")
        sid = lax.axis_index("s")
        wid = sid * 2 + cid
        _zero_fill(zbuf)
        def zrow(i, _):
            pltpu.sync_copy(zbuf, acc.at[pl.ds(sid * 640 + i * 64, 64)])
            return 0
        lax.fori_loop(0, 10, zrow, 0)
        plsc.subcore_barrier()

        base = wid * PER_W
        def chunk(c, _):
            off = base + c * CH
            pltpu.sync_copy(ind.at[pl.ds(off, CH)], idxv)
            pltpu.sync_copy(msg.at[pl.ds(off, CH)], mbuf)
            pltpu.sync_copy(mbuf, acc.at[idxv], add=True)
            return 0
        lax.fori_loop(0, NCHUNK, chunk, 0)
        plsc.subcore_barrier()
        pltpu.sync_copy(acc.at[pl.ds(sid * 640, 640)],
                        out.at[cid, pl.ds(sid * 640, 640)])

    return k(msgs, idx)


def _scatter_big(msgs, idx):
    mesh = plsc.VectorSubcoreMesh(**_MESH)

    @functools.partial(
        pl.kernel,
        out_type=jax.ShapeDtypeStruct((EP, D), jnp.float32),
        mesh=mesh,
        scratch_types=[
            pltpu.VMEM((CH,), jnp.int32),
            pltpu.VMEM((CH,), jnp.int32),
            pltpu.VMEM((CH, FS), jnp.float32),
            pltpu.VMEM((64, FS), jnp.float32),
            pltpu.VMEM_SHARED((SP_ROWS, FS), jnp.float32),
        ],
        compiler_params=_SC_PARAMS,
    )
    def k(msg, ind, out, idxv, clampv, mbuf, zbuf, acc):
        cid = lax.axis_index("c")
        sid = lax.axis_index("s")
        _zero_fill(zbuf)

        for ph in range(8):
            rp, fq = ph // 4, ph % 4
            @pl.when(cid == (fq % 2))
            def _phase():
                lo = rp * RHALF
                def zrow(i, _):
                    pltpu.sync_copy(zbuf, acc.at[pl.ds(sid * 5184 + i * 64, 64)])
                    return 0
                lax.fori_loop(0, 5184 // 64, zrow, 0)
                plsc.subcore_barrier()

                tbase = sid * (EP // 16)
                def chunk(c, _):
                    off = tbase + c * CH
                    pltpu.sync_copy(ind.at[pl.ds(off, CH)], idxv)
                    for v in range(CH // 16):
                        lv = idxv[pl.ds(v * 16, 16)]
                        il = lv - lo
                        ok = (il >= 0) & (il < RHALF)
                        clampv[pl.ds(v * 16, 16)] = jnp.where(ok, il, DUMP)
                    pltpu.sync_copy(msg.at[pl.ds(off, CH), pl.ds(fq * FS, FS)],
                                    mbuf)
                    pltpu.sync_copy(mbuf, acc.at[clampv], add=True)
                    return 0
                lax.fori_loop(0, (EP // 16) // CH, chunk, 0)
                plsc.subcore_barrier()
                pltpu.sync_copy(
                    acc.at[pl.ds(sid * (RHALF // 16), RHALF // 16)],
                    out.at[pl.ds(lo + sid * (RHALF // 16), RHALF // 16),
                           pl.ds(fq * FS, FS)])
                plsc.subcore_barrier()

    return k(msgs, idx)


def _embed_kernel(x_ref, w_ref, b_ref, o_ref):
    o_ref[...] = x_ref[...] @ w_ref[...] + b_ref[...]


def _tc_embed(x, w, b, rows_out, blk=512):
    din = x.shape[1]
    grid = rows_out // blk if rows_out % blk == 0 else rows_out // blk + 1
    return pl.pallas_call(
        _embed_kernel,
        grid=(grid,),
        in_specs=[
            pl.BlockSpec((blk, din), lambda i: (i, 0)),
            pl.BlockSpec((din, D), lambda i: (0, 0)),
            pl.BlockSpec((1, D), lambda i: (0, 0)),
        ],
        out_specs=pl.BlockSpec((blk, D), lambda i: (i, 0)),
        out_shape=jax.ShapeDtypeStruct((rows_out, D), jnp.float32),
    )(x, w, b.reshape(1, D))


def _silu(x):
    return x * (1.0 / (1.0 + jnp.exp(-x)))


def _msg_kernel(nblk, g_ref, ea_ref, w1_ref, b1_ref, w2_ref, b2_ref,
                w3_ref, b3_ref, wg2_ref, bg2_ref, o_ref, *, extra=None):
    ea = ea_ref[...]
    if extra is not None:
        ea = ea + extra[...]
    cc = jnp.concatenate([g_ref[0], g_ref[1], ea], axis=1)
    t = cc @ w1_ref[...] + b1_ref[...]
    z = _silu(t[:, :D])
    zg = _silu(t[:, D:])
    z2 = _silu(z @ w2_ref[...] + b2_ref[...])
    m = z2 @ w3_ref[...] + b3_ref[...]
    gate = 1.0 / (1.0 + jnp.exp(-(jnp.sum(zg * wg2_ref[...], axis=1,
                                          keepdims=True) + bg2_ref[...])))
    msg = gate * m
    rid = nblk * 512 + lax.broadcasted_iota(jnp.int32, (512, 1), 0)
    o_ref[...] = jnp.where(rid < E, msg, 0.0)


def _msg_kernel_plain(g, ea, w1, b1, w2, b2, w3, b3, wg2, bg2, o):
    _msg_kernel(pl.program_id(0), g, ea, w1, b1, w2, b2, w3, b3,
                wg2, bg2, o)


def _msg_kernel_scat(g, ea, sc, w1, b1, w2, b2, w3, b3, wg2, bg2, o):
    _msg_kernel(pl.program_id(0), g, ea, w1, b1, w2, b2, w3, b3,
                wg2, bg2, o, extra=sc)


def _tc_msg(g, ea, p, scat=None):
    w1 = jnp.concatenate(
        [jnp.concatenate([p["node1"]["w"][:D], p["gate1"]["w"][:D]], axis=1),
         jnp.concatenate([p["node1"]["w"][D:2 * D], p["gate1"]["w"][D:2 * D]], axis=1),
         jnp.concatenate([p["node1"]["w"][2 * D:], p["gate1"]["w"][2 * D:]], axis=1)],
        axis=0)
    b1 = jnp.concatenate([p["node1"]["b"], p["gate1"]["b"]]).reshape(1, 2 * D)
    blkspec = pl.BlockSpec((512, D), lambda i: (i, 0))
    gspec = pl.BlockSpec((2, 512, D), lambda i: (0, i, 0))
    wspec = lambda r, c: pl.BlockSpec((r, c), lambda i: (0, 0))
    ins = [g, ea]
    specs = [gspec, blkspec]
    kern = _msg_kernel_plain
    if scat is not None:
        ins.append(scat)
        specs.append(blkspec)
        kern = _msg_kernel_scat
    ins += [w1, b1, p["node2"]["w"], p["node2"]["b"].reshape(1, D),
            p["node3"]["w"], p["node3"]["b"].reshape(1, D),
            p["gate2"]["w"].reshape(1, D), p["gate2"]["b"].reshape(1, 1)]
    specs += [wspec(192, 128), wspec(1, 128), wspec(D, D), wspec(1, D),
              wspec(D, D), wspec(1, D), wspec(1, D), wspec(1, 1)]
    return pl.pallas_call(
        kern,
        grid=(EP // 512,),
        in_specs=specs,
        out_specs=blkspec,
        out_shape=jax.ShapeDtypeStruct((EP, D), jnp.float32),
    )(*ins)


def _hnew_kernel(h_ref, p_ref, o_ref):
    o_ref[...] = h_ref[...] + p_ref[0] + p_ref[1]


def _tc_hnew(h, parts):
    return pl.pallas_call(
        _hnew_kernel,
        grid=(20,),
        in_specs=[
            pl.BlockSpec((512, D), lambda i: (i, 0)),
            pl.BlockSpec((2, 512, D), lambda i: (0, i, 0)),
        ],
        out_specs=pl.BlockSpec((512, D), lambda i: (i, 0)),
        out_shape=jax.ShapeDtypeStruct((N, D), jnp.float32),
    )(h, parts)


def _eupd_kernel(g_ref, ea_ref, w1_ref, b1_ref, w2_ref, b2_ref, o_ref):
    cc = jnp.concatenate([g_ref[1], g_ref[0], ea_ref[...]], axis=1)
    t = _silu(cc @ w1_ref[...] + b1_ref[...])
    o_ref[...] = ea_ref[...] + t @ w2_ref[...] + b2_ref[...]


def _tc_eupd(g, ea, p):
    blkspec = pl.BlockSpec((512, D), lambda i: (i, 0))
    gspec = pl.BlockSpec((2, 512, D), lambda i: (0, i, 0))
    wspec = lambda r, c: pl.BlockSpec((r, c), lambda i: (0, 0))
    return pl.pallas_call(
        _eupd_kernel,
        grid=(EP // 512,),
        in_specs=[gspec, blkspec, wspec(192, D), wspec(1, D),
                  wspec(D, D), wspec(1, D)],
        out_specs=blkspec,
        out_shape=jax.ShapeDtypeStruct((EP, D), jnp.float32),
    )(g, ea, p["edge1"]["w"], p["edge1"]["b"].reshape(1, D),
      p["edge2"]["w"], p["edge2"]["b"].reshape(1, D))


def _final_kernel(h_ref, p_ref, u_ref, gw_ref, gb_ref, w1_ref, b1_ref,
                  w2_ref, b2_ref, o_ref):
    hsum = jnp.sum(h_ref[...], axis=0, keepdims=True)
    psum = jnp.sum(p_ref[0, :N] + p_ref[1, :N], axis=0, keepdims=True)
    pool = (hsum + psum) * (1.0 / N)
    ue = u_ref[...] @ gw_ref[...] + gb_ref[...]
    comb = jnp.concatenate([pool, ue], axis=1)
    z = _silu(comb @ w1_ref[...] + b1_ref[...])
    o_ref[...] = z @ w2_ref[...] + b2_ref[...]


def _tc_final(h1, parts, u2, params):
    return pl.pallas_call(
        _final_kernel,
        out_shape=jax.ShapeDtypeStruct((1, 1), jnp.float32),
    )(h1, parts, u2,
      params["global_embed"]["w"], params["global_embed"]["b"].reshape(1, D),
      params["out1"]["w"], params["out1"]["b"].reshape(1, D),
      params["out2"]["w"], params["out2"]["b"].reshape(1, 1))


def kernel(x, edge_index, edge_attr, line_graph_edge_index,
           line_graph_edge_attr, u, batch, params):
    src = edge_index[0]
    dst = edge_index[1]
    pad_e = jnp.zeros((EP - E,), jnp.int32)
    idx_a = jnp.stack([jnp.concatenate([dst, pad_e]),
                       jnp.concatenate([src, pad_e])])
    idx_l = jnp.stack([jnp.concatenate([line_graph_edge_index[1], pad_e]),
                       jnp.concatenate([line_graph_edge_index[0], pad_e])])
    ea_pad = jnp.concatenate(
        [edge_attr, jnp.zeros((EP - E, edge_attr.shape[1]), jnp.float32)])
    lea_pad = jnp.concatenate(
        [line_graph_edge_attr, jnp.zeros((EP - E, D), jnp.float32)])
    u2 = u.reshape(1, -1)

    l0, l1 = params["layers"][0], params["layers"][1]
    gat = _make_gather2()

    h0 = _tc_embed(x, params["node_embed"]["w"], params["node_embed"]["b"], N)
    e0 = _tc_embed(ea_pad, params["edge_embed"]["w"], params["edge_embed"]["b"], EP)

    g1 = gat(h0, idx_a)
    m1 = _tc_msg(g1, e0, l0["atom"])
    p1 = _scatter_small(m1, idx_a[0])
    h1 = _tc_hnew(h0, p1)
    g2 = gat(h1, idx_a)
    e1 = _tc_eupd(g2, e0, l0["atom"])

    g3 = gat(e1, idx_l)
    m2 = _tc_msg(g3, lea_pad, l0["edge"])
    s2 = _scatter_big(m2, idx_l[0])

    m3 = _tc_msg(g2, e1, l1["atom"], scat=s2)
    p2 = _scatter_small(m3, idx_a[0])

    return _tc_final(h1, p2, u2, params)


# speedup vs baseline: 1.8996x; 1.5670x over previous
import functools

import jax
import jax.numpy as jnp
from jax import lax
from jax.experimental import pallas as pl
from jax.experimental.pallas import tpu as pltpu
from jax.experimental.pallas import tpu_sc as plsc

N, E = 10000, 160000
D = 64
EP = 161280
NW = 32
PER_W = EP // NW
CH = 240
NCHUNK = PER_W // CH

NPAD = 10240
FS = 16
RHALF = EP // 2
SP_ROWS = 80896
DUMP = 80700
GRID_E = 313

_MESH = dict(core_axis_name="c", subcore_axis_name="s")
_SC_PARAMS = pltpu.CompilerParams(use_tc_tiling_on_sc=False)


def _zero_fill(zbuf):
    z = jnp.zeros((16,), jnp.float32)
    nv = zbuf.shape[1] // 16
    def row(r, _):
        for v in range(nv):
            zbuf[r, pl.ds(v * 16, 16)] = z
        return 0
    lax.fori_loop(0, zbuf.shape[0], row, 0)


def _gather2(table, idx_d, idx_s):
    mesh = plsc.VectorSubcoreMesh(**_MESH)

    @functools.partial(
        pl.kernel,
        out_type=jax.ShapeDtypeStruct((EP, 2 * D), jnp.float32),
        mesh=mesh,
        scratch_types=[
            pltpu.VMEM((PER_W,), jnp.int32),
            pltpu.VMEM((PER_W,), jnp.int32),
            pltpu.VMEM((CH, D), jnp.float32),
            pltpu.VMEM((CH, D), jnp.float32),
            pltpu.VMEM((CH, D), jnp.float32),
            pltpu.VMEM((CH, D), jnp.float32),
            pltpu.SemaphoreType.DMA,
            pltpu.SemaphoreType.DMA,
            pltpu.SemaphoreType.DMA,
            pltpu.SemaphoreType.DMA,
        ],
        compiler_params=_SC_PARAMS,
    )
    def k(tab, ind_d, ind_s, out, idx_d, idx_s, gd0, gd1, gs0, gs1,
          sg0, sg1, so0, so1):
        wid = lax.axis_index("s") * 2 + lax.axis_index("c")
        base = wid * PER_W
        pltpu.sync_copy(ind_d.at[pl.ds(base, PER_W)], idx_d)
        pltpu.sync_copy(ind_s.at[pl.ds(base, PER_W)], idx_s)
        sg = (sg0, sg1)
        so = (so0, so1)
        gbd = (gd0, gd1)
        gbs = (gs0, gs1)

        def fire(c):
            s = c % 2
            d0 = pltpu.async_copy(
                tab.at[idx_d.at[pl.ds(c * CH, CH)]], gbd[s], sg[s])
            d1 = pltpu.async_copy(
                tab.at[idx_s.at[pl.ds(c * CH, CH)]], gbs[s], sg[s])
            return d0, d1

        pend_g = fire(0)
        pend_o = [None, None]
        for c in range(NCHUNK):
            s = c % 2
            pend_g[0].wait()
            pend_g[1].wait()
            if c + 1 < NCHUNK:
                if pend_o[1 - s] is not None:
                    for d in pend_o[1 - s]:
                        d.wait()
                pend_g = fire(c + 1)
            row0 = base + c * CH
            pend_o[s] = (
                pltpu.async_copy(
                    gbd[s], out.at[pl.ds(row0, CH), pl.ds(0, D)], so[s]),
                pltpu.async_copy(
                    gbs[s], out.at[pl.ds(row0, CH), pl.ds(D, D)], so[s]),
            )
        for ds_ in pend_o:
            if ds_ is not None:
                for d in ds_:
                    d.wait()

    return k(table, idx_d, idx_s)


def _scatter_small(msgs, idx):
    mesh = plsc.VectorSubcoreMesh(**_MESH)

    @functools.partial(
        pl.kernel,
        out_type=jax.ShapeDtypeStruct((2, NPAD, 2 * D), jnp.float32),
        mesh=mesh,
        scratch_types=[
            pltpu.VMEM((PER_W,), jnp.int32),
            pltpu.VMEM((CH,), jnp.int32),
            pltpu.VMEM((CH,), jnp.int32),
            pltpu.VMEM((CH, D), jnp.float32),
            pltpu.VMEM((CH, D), jnp.float32),
            pltpu.VMEM((64, D), jnp.float32),
            pltpu.VMEM_SHARED((NPAD, D), jnp.float32),
            pltpu.SemaphoreType.DMA,
            pltpu.SemaphoreType.DMA,
            pltpu.SemaphoreType.DMA,
            pltpu.SemaphoreType.DMA,
        ],
        compiler_params=_SC_PARAMS,
    )
    def k(msg, ind, out, idxall, idxw0, idxw1, mbuf0, mbuf1, zbuf, acc,
          sm0, sm1, ss0, ss1):
        idxw = (idxw0, idxw1)
        mbuf = (mbuf0, mbuf1)
        cid = lax.axis_index("c")
        sid = lax.axis_index("s")
        wid = sid * 2 + cid
        base = wid * PER_W
        pltpu.sync_copy(ind.at[pl.ds(base, PER_W)], idxall)
        _zero_fill(zbuf)
        def zrow(i, _):
            pltpu.sync_copy(zbuf, acc.at[pl.ds(sid * 640 + i * 64, 64)])
            return 0
        lax.fori_loop(0, 10, zrow, 0)
        plsc.subcore_barrier()

        sm = (sm0, sm1)
        ss = (ss0, ss1)

        def fire(c):
            s = c % 2
            return pltpu.async_copy(
                msg.at[pl.ds(base + c * CH, CH), pl.ds(0, D)], mbuf[s], sm[s])

        pend_m = fire(0)
        pend_s = [None, None]
        for c in range(NCHUNK):
            s = c % 2
            for v in range(CH // 16):
                idxw[s][pl.ds(v * 16, 16)] = idxall[pl.ds(c * CH + v * 16, 16)]
            pend_m.wait()
            if c + 1 < NCHUNK:
                if pend_s[1 - s] is not None:
                    pend_s[1 - s].wait()
                    pend_s[1 - s] = None
                pend_m = fire(c + 1)
            pend_s[s] = pltpu.async_copy(
                mbuf[s], acc.at[idxw[s]], ss[s], add=True)
        for d in pend_s:
            if d is not None:
                d.wait()
        plsc.subcore_barrier()
        pltpu.sync_copy(acc.at[pl.ds(sid * 640, 640)],
                        out.at[cid, pl.ds(sid * 640, 640), pl.ds(0, D)])

    return k(msgs, idx)


def _scatter_big(msgs, idx):
    mesh = plsc.VectorSubcoreMesh(**_MESH)
    TCH = EP // 16
    NCH2 = TCH // CH

    @functools.partial(
        pl.kernel,
        out_type=jax.ShapeDtypeStruct((EP, 2 * D), jnp.float32),
        mesh=mesh,
        scratch_types=[
            pltpu.VMEM((TCH,), jnp.int32),
            pltpu.VMEM((CH,), jnp.int32),
            pltpu.VMEM((CH,), jnp.int32),
            pltpu.VMEM((CH, FS), jnp.float32),
            pltpu.VMEM((CH, FS), jnp.float32),
            pltpu.VMEM((64, FS), jnp.float32),
            pltpu.VMEM_SHARED((SP_ROWS, FS), jnp.float32),
            pltpu.SemaphoreType.DMA,
            pltpu.SemaphoreType.DMA,
            pltpu.SemaphoreType.DMA,
            pltpu.SemaphoreType.DMA,
        ],
        compiler_params=_SC_PARAMS,
    )
    def k(msg, ind, out, idxall, idxw0, idxw1, mbuf0, mbuf1, zbuf, acc,
          sm0, sm1, ss0, ss1):
        idxw = (idxw0, idxw1)
        mbuf = (mbuf0, mbuf1)
        cid = lax.axis_index("c")
        sid = lax.axis_index("s")
        pltpu.sync_copy(ind.at[pl.ds(sid * TCH, TCH)], idxall)
        _zero_fill(zbuf)
        sm = (sm0, sm1)
        ss = (ss0, ss1)

        for rp in range(2):
            for fq in range(4):
                @pl.when(cid == (fq % 2))
                def _phase(rp=rp, fq=fq):
                    lo = rp * RHALF
                    def zrow(i, _):
                        pltpu.sync_copy(
                            zbuf, acc.at[pl.ds(sid * 5056 + i * 64, 64)])
                        return 0
                    lax.fori_loop(0, 5056 // 64, zrow, 0)
                    plsc.subcore_barrier()

                    def fire_load(c, s):
                        pltpu.async_copy(
                            msg.at[pl.ds(sid * TCH + c * CH, CH),
                                   pl.ds(fq * FS, FS)],
                            mbuf[s], sm[s])

                    def wait_load(s):
                        pltpu.make_async_copy(
                            msg.at[pl.ds(0, CH), pl.ds(0, FS)],
                            mbuf[s], sm[s]).wait()

                    def wait_scat(s):
                        pltpu.make_async_copy(
                            mbuf[s], acc.at[idxw[s]], ss[s]).wait()

                    fire_load(0, 0)

                    def body(c2, carry):
                        for par in range(2):
                            c = c2 * 2 + par
                            s = par
                            for v in range(CH // 16):
                                lv = idxall[pl.ds(c * CH + v * 16, 16)]
                                il = lv - lo
                                ok = (il >= 0) & (il < RHALF)
                                idxw[s][pl.ds(v * 16, 16)] = (
                                    jnp.where(ok, il, DUMP))
                            wait_load(s)
                            @pl.when(c >= 1)
                            def _ws(s2=1 - s):
                                wait_scat(s2)
                            @pl.when(c + 1 < NCH2)
                            def _f(c=c, s2=1 - s):
                                fire_load(c + 1, s2)
                            pltpu.async_copy(
                                mbuf[s], acc.at[idxw[s]], ss[s],
                                add=True)
                        return carry
                    lax.fori_loop(0, NCH2 // 2, body, 0)
                    wait_scat(1)
                    plsc.subcore_barrier()
                    pltpu.sync_copy(
                        acc.at[pl.ds(sid * (RHALF // 16), RHALF // 16)],
                        out.at[pl.ds(lo + sid * (RHALF // 16), RHALF // 16),
                               pl.ds(fq * FS, FS)])
                    plsc.subcore_barrier()

    return k(msgs, idx)


def _silu(x):
    return x * (1.0 / (1.0 + jnp.exp(-x)))


def _embed_kernel(x_ref, w_ref, b_ref, o_ref):
    o_ref[...] = x_ref[...] @ w_ref[...] + b_ref[...]


def _tc_embed(x, w, b, rows_out, blk):
    din = x.shape[1]
    grid = (rows_out + blk - 1) // blk
    return pl.pallas_call(
        _embed_kernel,
        grid=(grid,),
        in_specs=[
            pl.BlockSpec((blk, din), lambda i: (i, 0)),
            pl.BlockSpec((din, D), lambda i: (0, 0)),
            pl.BlockSpec((1, D), lambda i: (0, 0)),
        ],
        out_specs=pl.BlockSpec((blk, D), lambda i: (i, 0)),
        out_shape=jax.ShapeDtypeStruct((rows_out, D), jnp.float32),
    )(x, w, b.reshape(1, D))


def _msg_body(g, ea, wd, ws, we, b1, w2, b2, w3, b3, wg2, bg2, o_ref):
    t = g[:, :D] @ wd[...] + g[:, D:] @ ws[...] + ea @ we[...] + b1[...]
    z = _silu(t[:, :D])
    zg = _silu(t[:, D:])
    z2 = _silu(z @ w2[...] + b2[...])
    m = z2 @ w3[...] + b3[...]
    gate = 1.0 / (1.0 + jnp.exp(-(zg @ wg2[...] + bg2[...])))
    o_ref[:, :D] = gate * m
    o_ref[:, D:] = jnp.zeros((o_ref.shape[0], D), jnp.float32)


def _msg_kernel_plain(g, ea, wd, ws, we, b1, w2, b2, w3, b3, wg2, bg2, o):
    _msg_body(g[...], ea[...], wd, ws, we, b1, w2, b2, w3, b3, wg2, bg2, o)


def _msg_kernel_scat(g, ea, sc, wd, ws, we, b1, w2, b2, w3, b3, wg2, bg2, o):
    _msg_body(g[...], ea[...] + sc[:, :D], wd, ws, we, b1, w2, b2, w3, b3,
              wg2, bg2, o)


def _tc_msg(g, ea, p, scat=None):
    wd = jnp.concatenate([p["node1"]["w"][:D], p["gate1"]["w"][:D]], axis=1)
    ws = jnp.concatenate([p["node1"]["w"][D:2 * D],
                          p["gate1"]["w"][D:2 * D]], axis=1)
    we = jnp.concatenate([p["node1"]["w"][2 * D:],
                          p["gate1"]["w"][2 * D:]], axis=1)
    b1 = jnp.concatenate([p["node1"]["b"], p["gate1"]["b"]]).reshape(1, 2 * D)
    espec = pl.BlockSpec((512, ea.shape[1]), lambda i: (i, 0))
    gspec = pl.BlockSpec((512, 2 * D), lambda i: (i, 0))
    wspec = lambda r, c: pl.BlockSpec((r, c), lambda i: (0, 0))
    ins = [g, ea]
    specs = [gspec, espec]
    kern = _msg_kernel_plain
    if scat is not None:
        ins.append(scat)
        specs.append(pl.BlockSpec((512, 2 * D), lambda i: (i, 0)))
        kern = _msg_kernel_scat
    ins += [wd, ws, we, b1, p["node2"]["w"], p["node2"]["b"].reshape(1, D),
            p["node3"]["w"], p["node3"]["b"].reshape(1, D),
            p["gate2"]["w"], p["gate2"]["b"].reshape(1, 1)]
    specs += [wspec(D, 2 * D), wspec(D, 2 * D), wspec(D, 2 * D),
              wspec(1, 2 * D), wspec(D, D), wspec(1, D), wspec(D, D),
              wspec(1, D), wspec(D, 1), wspec(1, 1)]
    return pl.pallas_call(
        kern,
        grid=(GRID_E,),
        in_specs=specs,
        out_specs=pl.BlockSpec((512, 2 * D), lambda i: (i, 0)),
        out_shape=jax.ShapeDtypeStruct((EP, 2 * D), jnp.float32),
    )(*ins)


def _hnew_kernel(h_ref, p_ref, o_ref):
    o_ref[...] = h_ref[...] + p_ref[0, :, :D] + p_ref[1, :, :D]


def _tc_hnew(h, parts):
    return pl.pallas_call(
        _hnew_kernel,
        grid=(20,),
        in_specs=[
            pl.BlockSpec((512, D), lambda i: (i, 0)),
            pl.BlockSpec((2, 512, 2 * D), lambda i: (0, i, 0)),
        ],
        out_specs=pl.BlockSpec((512, D), lambda i: (i, 0)),
        out_shape=jax.ShapeDtypeStruct((N, D), jnp.float32),
    )(h, parts)


def _eupd_kernel(g_ref, ea_ref, w1s_ref, w1d_ref, w1e_ref, b1_ref, w2_ref,
                 b2_ref, o_ref):
    g = g_ref[...]
    ea = ea_ref[...]
    t = _silu(g[:, D:] @ w1s_ref[...] + g[:, :D] @ w1d_ref[...]
              + ea @ w1e_ref[...] + b1_ref[...])
    o_ref[...] = ea + t @ w2_ref[...] + b2_ref[...]


def _tc_eupd(g, ea, p):
    espec = pl.BlockSpec((512, D), lambda i: (i, 0))
    gspec = pl.BlockSpec((512, 2 * D), lambda i: (i, 0))
    wspec = lambda r, c: pl.BlockSpec((r, c), lambda i: (0, 0))
    w1 = p["edge1"]["w"]
    return pl.pallas_call(
        _eupd_kernel,
        grid=(GRID_E,),
        in_specs=[gspec, espec, wspec(D, D), wspec(D, D), wspec(D, D),
                  wspec(1, D), wspec(D, D), wspec(1, D)],
        out_specs=espec,
        out_shape=jax.ShapeDtypeStruct((EP, D), jnp.float32),
    )(g, ea, w1[:D], w1[D:2 * D], w1[2 * D:], p["edge1"]["b"].reshape(1, D),
      p["edge2"]["w"], p["edge2"]["b"].reshape(1, D))


def _final_kernel(h_ref, p_ref, u_ref, gw_ref, gb_ref, w1_ref, b1_ref,
                  w2_ref, b2_ref, o_ref):
    hsum = jnp.sum(h_ref[...], axis=0, keepdims=True)
    psum = jnp.sum(p_ref[0, :N, :D] + p_ref[1, :N, :D], axis=0, keepdims=True)
    pool = (hsum + psum) * (1.0 / N)
    ue = u_ref[...] @ gw_ref[...] + gb_ref[...]
    comb = jnp.concatenate([pool, ue], axis=1)
    z = _silu(comb @ w1_ref[...] + b1_ref[...])
    o_ref[...] = z @ w2_ref[...] + b2_ref[...]


def _tc_final(h1, parts, u2, params):
    return pl.pallas_call(
        _final_kernel,
        out_shape=jax.ShapeDtypeStruct((1, 1), jnp.float32),
    )(h1, parts, u2,
      params["global_embed"]["w"], params["global_embed"]["b"].reshape(1, D),
      params["out1"]["w"], params["out1"]["b"].reshape(1, D),
      params["out2"]["w"], params["out2"]["b"].reshape(1, 1))


def kernel(x, edge_index, edge_attr, line_graph_edge_index,
           line_graph_edge_attr, u, batch, params):
    pad0 = jnp.zeros((EP - E,), jnp.int32)
    src_g = jnp.concatenate([edge_index[0], pad0])
    dst_g = jnp.concatenate([edge_index[1], pad0])
    dst_s = jnp.concatenate([edge_index[1],
                             jnp.full((EP - E,), NPAD - 1, jnp.int32)])
    lsrc_g = jnp.concatenate([line_graph_edge_index[0], pad0])
    ldst_g = jnp.concatenate([line_graph_edge_index[1], pad0])
    ldst_s = jnp.concatenate([line_graph_edge_index[1],
                              jnp.full((EP - E,), 2 * EP, jnp.int32)])
    u2 = u.reshape(1, -1)

    l0, l1 = params["layers"][0], params["layers"][1]

    h0 = _tc_embed(x, params["node_embed"]["w"], params["node_embed"]["b"],
                   N, 2048)
    e0 = _tc_embed(edge_attr, params["edge_embed"]["w"],
                   params["edge_embed"]["b"], EP, 4096)

    g1 = _gather2(h0, dst_g, src_g)
    m1 = _tc_msg(g1, e0, l0["atom"])
    p1 = _scatter_small(m1, dst_s)
    h1 = _tc_hnew(h0, p1)
    g2 = _gather2(h1, dst_g, src_g)
    e1 = _tc_eupd(g2, e0, l0["atom"])

    g3 = _gather2(e1, ldst_g, lsrc_g)
    m2 = _tc_msg(g3, line_graph_edge_attr, l0["edge"])
    s2 = _scatter_big(m2, ldst_s)

    m3 = _tc_msg(g2, e1, l1["atom"], scat=s2)
    p2 = _scatter_small(m3, dst_s)

    return _tc_final(h1, p2, u2, params)


# speedup vs baseline: 2.3071x; 1.2145x over previous
import functools

import jax
import jax.numpy as jnp
from jax import lax
from jax.experimental import pallas as pl
from jax.experimental.pallas import tpu as pltpu
from jax.experimental.pallas import tpu_sc as plsc

N, E = 10000, 160000
D = 64
EP = 161280
NW = 32
PER_W = EP // NW
CH = 336
NCHUNK = PER_W // CH

NPAD = 10240
FS = 16
RHALF = EP // 2
SP_ROWS = 80896
DUMP = 80700
BLK = 1024
GRID_E = 157

_MESH = dict(core_axis_name="c", subcore_axis_name="s")
_SC_PARAMS = pltpu.CompilerParams(use_tc_tiling_on_sc=False)


def _zero_fill(zbuf):
    z = jnp.zeros((16,), jnp.float32)
    nv = zbuf.shape[1] // 16
    def row(r, _):
        for v in range(nv):
            zbuf[r, pl.ds(v * 16, 16)] = z
        return 0
    lax.fori_loop(0, zbuf.shape[0], row, 0)


def _gather2(table, idx_d, idx_s):
    mesh = plsc.VectorSubcoreMesh(**_MESH)

    @functools.partial(
        pl.kernel,
        out_type=jax.ShapeDtypeStruct((EP, 2 * D), jnp.float32),
        mesh=mesh,
        scratch_types=[
            pltpu.VMEM((PER_W,), jnp.int32),
            pltpu.VMEM((PER_W,), jnp.int32),
            pltpu.VMEM((CH, D), jnp.float32),
            pltpu.VMEM((CH, D), jnp.float32),
            pltpu.VMEM((CH, D), jnp.float32),
            pltpu.VMEM((CH, D), jnp.float32),
            pltpu.SemaphoreType.DMA,
            pltpu.SemaphoreType.DMA,
            pltpu.SemaphoreType.DMA,
            pltpu.SemaphoreType.DMA,
        ],
        compiler_params=_SC_PARAMS,
    )
    def k(tab, ind_d, ind_s, out, idx_d, idx_s, gd0, gd1, gs0, gs1,
          sg0, sg1, so0, so1):
        wid = lax.axis_index("s") * 2 + lax.axis_index("c")
        base = wid * PER_W
        pltpu.sync_copy(ind_d.at[pl.ds(base, PER_W)], idx_d)
        pltpu.sync_copy(ind_s.at[pl.ds(base, PER_W)], idx_s)
        sg = (sg0, sg1)
        so = (so0, so1)
        gbd = (gd0, gd1)
        gbs = (gs0, gs1)

        def fire(c):
            s = c % 2
            d0 = pltpu.async_copy(
                tab.at[idx_d.at[pl.ds(c * CH, CH)]], gbd[s], sg[s])
            d1 = pltpu.async_copy(
                tab.at[idx_s.at[pl.ds(c * CH, CH)]], gbs[s], sg[s])
            return d0, d1

        pend_g = fire(0)
        pend_o = [None, None]
        for c in range(NCHUNK):
            s = c % 2
            pend_g[0].wait()
            pend_g[1].wait()
            if c + 1 < NCHUNK:
                if pend_o[1 - s] is not None:
                    for d in pend_o[1 - s]:
                        d.wait()
                pend_g = fire(c + 1)
            row0 = base + c * CH
            pend_o[s] = (
                pltpu.async_copy(
                    gbd[s], out.at[pl.ds(row0, CH), pl.ds(0, D)], so[s]),
                pltpu.async_copy(
                    gbs[s], out.at[pl.ds(row0, CH), pl.ds(D, D)], so[s]),
            )
        for ds_ in pend_o:
            if ds_ is not None:
                for d in ds_:
                    d.wait()

    return k(table, idx_d, idx_s)


def _scatter_small(msgs, idx):
    mesh = plsc.VectorSubcoreMesh(**_MESH)

    @functools.partial(
        pl.kernel,
        out_type=jax.ShapeDtypeStruct((2, NPAD, 2 * D), jnp.float32),
        mesh=mesh,
        scratch_types=[
            pltpu.VMEM((PER_W,), jnp.int32),
            pltpu.VMEM((CH,), jnp.int32),
            pltpu.VMEM((CH,), jnp.int32),
            pltpu.VMEM((CH, D), jnp.float32),
            pltpu.VMEM((CH, D), jnp.float32),
            pltpu.VMEM((640, D), jnp.float32),
            pltpu.VMEM_SHARED((NPAD, D), jnp.float32),
            pltpu.SemaphoreType.DMA,
            pltpu.SemaphoreType.DMA,
            pltpu.SemaphoreType.DMA,
            pltpu.SemaphoreType.DMA,
        ],
        compiler_params=_SC_PARAMS,
    )
    def k(msg, ind, out, idxall, idxw0, idxw1, mbuf0, mbuf1, zbuf, acc,
          sm0, sm1, ss0, ss1):
        idxw = (idxw0, idxw1)
        mbuf = (mbuf0, mbuf1)
        cid = lax.axis_index("c")
        sid = lax.axis_index("s")
        wid = sid * 2 + cid
        base = wid * PER_W
        pltpu.sync_copy(ind.at[pl.ds(base, PER_W)], idxall)
        _zero_fill(zbuf)
        pltpu.sync_copy(zbuf, acc.at[pl.ds(sid * 640, 640)])
        plsc.subcore_barrier()

        sm = (sm0, sm1)
        ss = (ss0, ss1)

        def fire(c):
            s = c % 2
            return pltpu.async_copy(
                msg.at[pl.ds(base + c * CH, CH), pl.ds(0, D)], mbuf[s], sm[s])

        pend_m = fire(0)
        pend_s = [None, None]
        for c in range(NCHUNK):
            s = c % 2
            for v in range(CH // 16):
                idxw[s][pl.ds(v * 16, 16)] = idxall[pl.ds(c * CH + v * 16, 16)]
            pend_m.wait()
            if c + 1 < NCHUNK:
                if pend_s[1 - s] is not None:
                    pend_s[1 - s].wait()
                    pend_s[1 - s] = None
                pend_m = fire(c + 1)
            pend_s[s] = pltpu.async_copy(
                mbuf[s], acc.at[idxw[s]], ss[s], add=True)
        for d in pend_s:
            if d is not None:
                d.wait()
        plsc.subcore_barrier()
        pltpu.sync_copy(acc.at[pl.ds(sid * 640, 640)],
                        out.at[cid, pl.ds(sid * 640, 640), pl.ds(0, D)])

    return k(msgs, idx)


def _scatter_big(msgs, idx):
    mesh = plsc.VectorSubcoreMesh(**_MESH)
    TCH = EP // 16
    NCH2 = TCH // CH

    @functools.partial(
        pl.kernel,
        out_type=jax.ShapeDtypeStruct((EP, 2 * D), jnp.float32),
        mesh=mesh,
        scratch_types=[
            pltpu.VMEM((TCH,), jnp.int32),
            pltpu.VMEM((CH,), jnp.int32),
            pltpu.VMEM((CH,), jnp.int32),
            pltpu.VMEM((CH, FS), jnp.float32),
            pltpu.VMEM((CH, FS), jnp.float32),
            pltpu.VMEM((1024, FS), jnp.float32),
            pltpu.VMEM_SHARED((SP_ROWS, FS), jnp.float32),
            pltpu.SemaphoreType.DMA,
            pltpu.SemaphoreType.DMA,
            pltpu.SemaphoreType.DMA,
            pltpu.SemaphoreType.DMA,
        ],
        compiler_params=_SC_PARAMS,
    )
    def k(msg, ind, out, idxall, idxw0, idxw1, mbuf0, mbuf1, zbuf, acc,
          sm0, sm1, ss0, ss1):
        idxw = (idxw0, idxw1)
        mbuf = (mbuf0, mbuf1)
        cid = lax.axis_index("c")
        sid = lax.axis_index("s")
        pltpu.sync_copy(ind.at[pl.ds(sid * TCH, TCH)], idxall)
        _zero_fill(zbuf)
        sm = (sm0, sm1)
        ss = (ss0, ss1)

        for rp in range(2):
            for fq in range(4):
                @pl.when(cid == (fq % 2))
                def _phase(rp=rp, fq=fq):
                    lo = rp * RHALF
                    zd = []
                    for i in range(4):
                        zd.append(pltpu.async_copy(
                            zbuf, acc.at[pl.ds(sid * 5056 + i * 1024, 1024)],
                            sm[0]))
                    zd.append(pltpu.async_copy(
                        zbuf.at[pl.ds(0, 960)],
                        acc.at[pl.ds(sid * 5056 + 4096, 960)], sm[0]))
                    for d in zd:
                        d.wait()
                    plsc.subcore_barrier()

                    def fire_load(c, s):
                        pltpu.async_copy(
                            msg.at[pl.ds(sid * TCH + c * CH, CH),
                                   pl.ds(fq * FS, FS)],
                            mbuf[s], sm[s])

                    def wait_load(s):
                        pltpu.make_async_copy(
                            msg.at[pl.ds(0, CH), pl.ds(0, FS)],
                            mbuf[s], sm[s]).wait()

                    def wait_scat(s):
                        pltpu.make_async_copy(
                            mbuf[s], acc.at[idxw[s]], ss[s]).wait()

                    fire_load(0, 0)

                    def body(c2, carry):
                        for par in range(2):
                            c = c2 * 2 + par
                            s = par
                            for v in range(CH // 16):
                                lv = idxall[pl.ds(c * CH + v * 16, 16)]
                                il = lv - lo
                                ok = (il >= 0) & (il < RHALF)
                                idxw[s][pl.ds(v * 16, 16)] = (
                                    jnp.where(ok, il, DUMP))
                            wait_load(s)
                            @pl.when(c >= 1)
                            def _ws(s2=1 - s):
                                wait_scat(s2)
                            @pl.when(c + 1 < NCH2)
                            def _f(c=c, s2=1 - s):
                                fire_load(c + 1, s2)
                            pltpu.async_copy(
                                mbuf[s], acc.at[idxw[s]], ss[s],
                                add=True)
                        return carry
                    lax.fori_loop(0, NCH2 // 2, body, 0)
                    wait_scat(1)
                    plsc.subcore_barrier()
                    pltpu.sync_copy(
                        acc.at[pl.ds(sid * (RHALF // 16), RHALF // 16)],
                        out.at[pl.ds(lo + sid * (RHALF // 16), RHALF // 16),
                               pl.ds(fq * FS, FS)])
                    plsc.subcore_barrier()

    return k(msgs, idx)


def _silu(x):
    return x * (1.0 / (1.0 + jnp.exp(-x)))


def _embed_kernel(x_ref, w_ref, b_ref, o_ref):
    o_ref[...] = x_ref[...] @ w_ref[...] + b_ref[...]


def _tc_embed(x, w, b, rows_out, blk):
    din = x.shape[1]
    grid = (rows_out + blk - 1) // blk
    return pl.pallas_call(
        _embed_kernel,
        grid=(grid,),
        in_specs=[
            pl.BlockSpec((blk, din), lambda i: (i, 0)),
            pl.BlockSpec((din, D), lambda i: (0, 0)),
            pl.BlockSpec((1, D), lambda i: (0, 0)),
        ],
        out_specs=pl.BlockSpec((blk, D), lambda i: (i, 0)),
        out_shape=jax.ShapeDtypeStruct((rows_out, D), jnp.float32),
    )(x, w, b.reshape(1, D))


def _msg_body(g, ea, wd, ws, we, b1, w2, b2, w3, b3, wg2, bg2, o_ref):
    t = g[:, :D] @ wd[...] + g[:, D:] @ ws[...] + ea @ we[...] + b1[...]
    z = _silu(t[:, :D])
    zg = _silu(t[:, D:])
    z2 = _silu(z @ w2[...] + b2[...])
    m = z2 @ w3[...] + b3[...]
    gate = 1.0 / (1.0 + jnp.exp(-(zg @ wg2[...] + bg2[...])))
    o_ref[:, :D] = gate * m
    o_ref[:, D:] = jnp.zeros((o_ref.shape[0], D), jnp.float32)


def _msg_kernel_plain(g, ea, wd, ws, we, b1, w2, b2, w3, b3, wg2, bg2, o):
    _msg_body(g[...], ea[...], wd, ws, we, b1, w2, b2, w3, b3, wg2, bg2, o)


def _msg_kernel_scat(g, ea, sc, wd, ws, we, b1, w2, b2, w3, b3, wg2, bg2, o):
    _msg_body(g[...], ea[...] + sc[:, :D], wd, ws, we, b1, w2, b2, w3, b3,
              wg2, bg2, o)


def _tc_msg(g, ea, p, scat=None):
    wd = jnp.concatenate([p["node1"]["w"][:D], p["gate1"]["w"][:D]], axis=1)
    ws = jnp.concatenate([p["node1"]["w"][D:2 * D],
                          p["gate1"]["w"][D:2 * D]], axis=1)
    we = jnp.concatenate([p["node1"]["w"][2 * D:],
                          p["gate1"]["w"][2 * D:]], axis=1)
    b1 = jnp.concatenate([p["node1"]["b"], p["gate1"]["b"]]).reshape(1, 2 * D)
    espec = pl.BlockSpec((BLK, ea.shape[1]), lambda i: (i, 0))
    gspec = pl.BlockSpec((BLK, 2 * D), lambda i: (i, 0))
    wspec = lambda r, c: pl.BlockSpec((r, c), lambda i: (0, 0))
    ins = [g, ea]
    specs = [gspec, espec]
    kern = _msg_kernel_plain
    if scat is not None:
        ins.append(scat)
        specs.append(pl.BlockSpec((BLK, 2 * D), lambda i: (i, 0)))
        kern = _msg_kernel_scat
    ins += [wd, ws, we, b1, p["node2"]["w"], p["node2"]["b"].reshape(1, D),
            p["node3"]["w"], p["node3"]["b"].reshape(1, D),
            p["gate2"]["w"], p["gate2"]["b"].reshape(1, 1)]
    specs += [wspec(D, 2 * D), wspec(D, 2 * D), wspec(D, 2 * D),
              wspec(1, 2 * D), wspec(D, D), wspec(1, D), wspec(D, D),
              wspec(1, D), wspec(D, 1), wspec(1, 1)]
    return pl.pallas_call(
        kern,
        grid=(GRID_E,),
        in_specs=specs,
        out_specs=pl.BlockSpec((BLK, 2 * D), lambda i: (i, 0)),
        out_shape=jax.ShapeDtypeStruct((EP, 2 * D), jnp.float32),
    )(*ins)


def _hnew_kernel(h_ref, p_ref, o_ref):
    o_ref[...] = h_ref[...] + p_ref[0, :, :D] + p_ref[1, :, :D]


def _tc_hnew(h, parts):
    return pl.pallas_call(
        _hnew_kernel,
        grid=(20,),
        in_specs=[
            pl.BlockSpec((512, D), lambda i: (i, 0)),
            pl.BlockSpec((2, 512, 2 * D), lambda i: (0, i, 0)),
        ],
        out_specs=pl.BlockSpec((512, D), lambda i: (i, 0)),
        out_shape=jax.ShapeDtypeStruct((N, D), jnp.float32),
    )(h, parts)


def _eupd_kernel(g_ref, ea_ref, w1s_ref, w1d_ref, w1e_ref, b1_ref, w2_ref,
                 b2_ref, o_ref):
    g = g_ref[...]
    ea = ea_ref[...]
    t = _silu(g[:, D:] @ w1s_ref[...] + g[:, :D] @ w1d_ref[...]
              + ea @ w1e_ref[...] + b1_ref[...])
    o_ref[...] = ea + t @ w2_ref[...] + b2_ref[...]


def _tc_eupd(g, ea, p):
    espec = pl.BlockSpec((BLK, D), lambda i: (i, 0))
    gspec = pl.BlockSpec((BLK, 2 * D), lambda i: (i, 0))
    wspec = lambda r, c: pl.BlockSpec((r, c), lambda i: (0, 0))
    w1 = p["edge1"]["w"]
    return pl.pallas_call(
        _eupd_kernel,
        grid=(GRID_E,),
        in_specs=[gspec, espec, wspec(D, D), wspec(D, D), wspec(D, D),
                  wspec(1, D), wspec(D, D), wspec(1, D)],
        out_specs=espec,
        out_shape=jax.ShapeDtypeStruct((EP, D), jnp.float32),
    )(g, ea, w1[:D], w1[D:2 * D], w1[2 * D:], p["edge1"]["b"].reshape(1, D),
      p["edge2"]["w"], p["edge2"]["b"].reshape(1, D))


def _final_kernel(h_ref, p_ref, u_ref, gw_ref, gb_ref, w1_ref, b1_ref,
                  w2_ref, b2_ref, o_ref):
    hsum = jnp.sum(h_ref[...], axis=0, keepdims=True)
    psum = jnp.sum(p_ref[0, :N, :D] + p_ref[1, :N, :D], axis=0, keepdims=True)
    pool = (hsum + psum) * (1.0 / N)
    ue = u_ref[...] @ gw_ref[...] + gb_ref[...]
    comb = jnp.concatenate([pool, ue], axis=1)
    z = _silu(comb @ w1_ref[...] + b1_ref[...])
    o_ref[...] = z @ w2_ref[...] + b2_ref[...]


def _tc_final(h1, parts, u2, params):
    return pl.pallas_call(
        _final_kernel,
        out_shape=jax.ShapeDtypeStruct((1, 1), jnp.float32),
    )(h1, parts, u2,
      params["global_embed"]["w"], params["global_embed"]["b"].reshape(1, D),
      params["out1"]["w"], params["out1"]["b"].reshape(1, D),
      params["out2"]["w"], params["out2"]["b"].reshape(1, 1))


def kernel(x, edge_index, edge_attr, line_graph_edge_index,
           line_graph_edge_attr, u, batch, params):
    pad0 = jnp.zeros((EP - E,), jnp.int32)
    src_g = jnp.concatenate([edge_index[0], pad0])
    dst_g = jnp.concatenate([edge_index[1], pad0])
    dst_s = jnp.concatenate([edge_index[1],
                             jnp.full((EP - E,), NPAD - 1, jnp.int32)])
    lsrc_g = jnp.concatenate([line_graph_edge_index[0], pad0])
    ldst_g = jnp.concatenate([line_graph_edge_index[1], pad0])
    ldst_s = jnp.concatenate([line_graph_edge_index[1],
                              jnp.full((EP - E,), 2 * EP, jnp.int32)])
    u2 = u.reshape(1, -1)

    l0, l1 = params["layers"][0], params["layers"][1]

    h0 = _tc_embed(x, params["node_embed"]["w"], params["node_embed"]["b"],
                   N, 2048)
    e0 = _tc_embed(edge_attr, params["edge_embed"]["w"],
                   params["edge_embed"]["b"], EP, 4096)

    g1 = _gather2(h0, dst_g, src_g)
    m1 = _tc_msg(g1, e0, l0["atom"])
    p1 = _scatter_small(m1, dst_s)
    h1 = _tc_hnew(h0, p1)
    g2 = _gather2(h1, dst_g, src_g)
    e1 = _tc_eupd(g2, e0, l0["atom"])

    g3 = _gather2(e1, ldst_g, lsrc_g)
    m2 = _tc_msg(g3, line_graph_edge_attr, l0["edge"])
    s2 = _scatter_big(m2, ldst_s)

    m3 = _tc_msg(g2, e1, l1["atom"], scat=s2)
    p2 = _scatter_small(m3, dst_s)

    return _tc_final(h1, p2, u2, params)
